# Initial kernel scaffold; baseline (speedup 1.0000x reference)
import jax
import jax.numpy as jnp
from jax.experimental import pallas as pl


def kernel(x, edge_index, W1, b1, W2, b2):
    raise NotImplementedError("write your pallas kernel here")


# speedup vs baseline: 12.7816x; 12.7816x over previous
import functools

import jax
import jax.numpy as jnp
from jax import lax
from jax.experimental import pallas as pl
from jax.experimental.pallas import tpu as pltpu
from jax.experimental.pallas import tpu_sc as plsc

NC = 2
NS = 16
LANES = 16


def _mesh():
    return plsc.VectorSubcoreMesh(core_axis_name="c", subcore_axis_name="s")


def _splits(total, step):
    out = []
    off = 0
    while off < total:
        sz = min(step, total - off)
        out.append((off, sz))
        off += sz
    return out


def _make_deg_kernel(E, NP):
    per_tile = E // (NC * NS)
    BLK = 1024
    nfull = per_tile // BLK
    rem = per_tile - nfull * BLK
    tile_slice = NP // NS
    assert rem % LANES == 0 and tile_slice % 8 == 0

    @functools.partial(
        pl.kernel,
        mesh=_mesh(),
        out_type=jax.ShapeDtypeStruct((NC * NP,), jnp.float32),
        scratch_types=[
            pltpu.VMEM((BLK,), jnp.int32),
            pltpu.VMEM((8, 128), jnp.int32),
            pltpu.VMEM((128,), jnp.float32),
            pltpu.VMEM((1024,), jnp.float32),
            pltpu.VMEM((1024,), jnp.float32),
            pltpu.VMEM_SHARED((NP,), jnp.float32),
        ],
    )
    def deg_k(dst_hbm, deg_hbm, dstbuf, idx2d, ones_v, zero_v, bounce, acc_sh):
        c = lax.axis_index("c")
        s = lax.axis_index("s")
        wid = c * NS + s
        iota = lax.iota(jnp.int32, LANES)
        onev = jnp.full((LANES,), 1.0, dtype=jnp.float32)
        zerov = jnp.zeros((LANES,), jnp.float32)
        for j in range(128 // LANES):
            ones_v[pl.ds(j * LANES, LANES)] = onev
        for j in range(1024 // LANES):
            zero_v[pl.ds(j * LANES, LANES)] = zerov
        for off, sz in _splits(tile_slice, 1024):
            pltpu.sync_copy(zero_v.at[pl.ds(0, sz)],
                            acc_sh.at[pl.ds(s * tile_slice + off, sz)])
        plsc.subcore_barrier()

        def stage_to_2d(nrows):
            for j in range(nrows):
                for t in range(128 // LANES):
                    idx2d[j, pl.ds(t * LANES, LANES)] = (
                        dstbuf[pl.ds(j * 128 + t * LANES, LANES)])

        def block(b, carry):
            base = wid * per_tile + b * BLK
            pltpu.sync_copy(dst_hbm.at[pl.ds(base, BLK)], dstbuf)
            stage_to_2d(8)
            for j in range(8):
                pltpu.sync_copy(ones_v, acc_sh.at[idx2d.at[j]], add=True)
            return carry

        lax.fori_loop(0, nfull, block, 0)
        if rem:
            base = wid * per_tile + nfull * BLK
            pltpu.sync_copy(dst_hbm.at[pl.ds(base, rem)],
                            dstbuf.at[pl.ds(0, rem)])
            nrows_full = rem // 128
            tail = rem - nrows_full * 128
            stage_to_2d(nrows_full)
            nrows = nrows_full
            if tail:
                garbage = (NP - LANES) + iota
                for t in range(128 // LANES):
                    src = tail // LANES
                    if t < src:
                        idx2d[nrows_full, pl.ds(t * LANES, LANES)] = (
                            dstbuf[pl.ds(nrows_full * 128 + t * LANES, LANES)])
                    else:
                        idx2d[nrows_full, pl.ds(t * LANES, LANES)] = garbage
                nrows = nrows_full + 1
            for j in range(nrows):
                pltpu.sync_copy(ones_v, acc_sh.at[idx2d.at[j]], add=True)

        plsc.subcore_barrier()
        for off, sz in _splits(tile_slice, 1024):
            pltpu.sync_copy(acc_sh.at[pl.ds(s * tile_slice + off, sz)],
                            bounce.at[pl.ds(0, sz)])
            pltpu.sync_copy(bounce.at[pl.ds(0, sz)],
                            deg_hbm.at[pl.ds(c * NP + s * tile_slice + off, sz)])

    return deg_k


def _make_spmm_kernel(D, E, CR, KCH, OUTROWS):
    KPC = KCH // NC
    per_tile = E // NS
    BLK = 1024
    nfull = per_tile // BLK
    rem = per_tile - nfull * BLK
    B = 128
    CAP = B + 2 * LANES
    RS = CR // NS
    ZR = 8192 // D
    assert rem % LANES == 0 and RS % 8 == 0 and OUTROWS == KCH * CR

    @functools.partial(
        pl.kernel,
        mesh=_mesh(),
        out_type=jax.ShapeDtypeStruct((OUTROWS, D), jnp.float32),
        scratch_types=[
            pltpu.VMEM((BLK,), jnp.int32),
            pltpu.VMEM((BLK,), jnp.int32),
            pltpu.VMEM((CAP,), jnp.int32),
            pltpu.VMEM((CAP,), jnp.int32),
            pltpu.VMEM((B,), jnp.int32),
            pltpu.VMEM((B,), jnp.int32),
            pltpu.VMEM((B, D), jnp.float32),
            pltpu.VMEM((ZR, D), jnp.float32),
            pltpu.VMEM_SHARED((CR + 8, D), jnp.float32),
            pltpu.SemaphoreType.DMA,
        ],
    )
    def spmm_k(hp_hbm, src_hbm, dst_hbm, out_hbm, srcbuf, dstbuf, csrc, cdst,
               fsrc, fdst, rows_v, zero_v, acc_sh, sem):
        c = lax.axis_index("c")
        s = lax.axis_index("s")
        iota = lax.iota(jnp.int32, LANES)
        zero_i = jnp.full((LANES,), 0, jnp.int32)
        one_i = jnp.full((LANES,), 1, jnp.int32)
        cr_i = jnp.full((LANES,), CR, jnp.int32)
        zerov = jnp.zeros((LANES,), jnp.float32)

        def zero_fill(r, carry):
            for t in range(D // LANES):
                zero_v[r, pl.ds(t * LANES, LANES)] = zerov
            return carry
        lax.fori_loop(0, ZR, zero_fill, 0)

        def fire(cnt):
            cntv = lax.broadcast(cnt, (LANES,))
            for j in range(B // LANES):
                valid = (iota + (j * LANES)) < cntv
                sv = csrc[pl.ds(j * LANES, LANES)]
                dv = cdst[pl.ds(j * LANES, LANES)]
                fsrc[pl.ds(j * LANES, LANES)] = jnp.where(valid, sv, zero_i)
                fdst[pl.ds(j * LANES, LANES)] = jnp.where(valid, dv, cr_i)
            pltpu.async_copy(hp_hbm.at[fsrc], rows_v, sem).wait()
            pltpu.sync_copy(rows_v, acc_sh.at[fdst], add=True)

        def scan_groups(w, nsub, lo):
            lov = lax.broadcast(lo, (LANES,))

            def group(i, w):
                sv = srcbuf[pl.ds(i * LANES, LANES)]
                dv = dstbuf[pl.ds(i * LANES, LANES)]
                m = (dv >= lov) & (dv < lov + CR)
                sc = jnp.where(m, one_i, zero_i)
                for sh in (1, 2, 4, 8):
                    perm = jnp.where(iota >= sh, iota - sh, zero_i)
                    g2 = sc.at[perm].get(mode="promise_in_bounds")
                    sc = sc + jnp.where(iota >= sh, g2, zero_i)
                t = zero_i
                for sh in (8, 4, 2, 1):
                    cand = t + sh
                    probe = sc.at[cand - one_i].get(mode="promise_in_bounds")
                    t = jnp.where(probe <= iota, cand, t)
                csrc[pl.ds(w, LANES)] = sv.at[t].get(mode="promise_in_bounds")
                cdst[pl.ds(w, LANES)] = (
                    dv.at[t].get(mode="promise_in_bounds") - lov)
                w = w + sc[LANES - 1]

                @pl.when(w >= B)
                def _():
                    fire(B)
                    csrc[pl.ds(0, LANES)] = csrc[pl.ds(B, LANES)]
                    cdst[pl.ds(0, LANES)] = cdst[pl.ds(B, LANES)]

                return jnp.where(w >= B, w - B, w)
            return lax.fori_loop(0, nsub, group, w)

        for k in range(KPC):
            chunk = c * KPC + k
            lo = chunk * CR
            plsc.subcore_barrier()
            for off, sz in _splits(RS, ZR):
                pltpu.sync_copy(zero_v.at[pl.ds(0, sz)],
                                acc_sh.at[pl.ds(s * RS + off, sz)])
            plsc.subcore_barrier()

            def block(b, w):
                base = s * per_tile + b * BLK
                pltpu.sync_copy(src_hbm.at[pl.ds(base, BLK)], srcbuf)
                pltpu.sync_copy(dst_hbm.at[pl.ds(base, BLK)], dstbuf)
                return scan_groups(w, BLK // LANES, lo)

            w = lax.fori_loop(0, nfull, block, jnp.int32(0))
            if rem:
                base = s * per_tile + nfull * BLK
                pltpu.sync_copy(src_hbm.at[pl.ds(base, rem)],
                                srcbuf.at[pl.ds(0, rem)])
                pltpu.sync_copy(dst_hbm.at[pl.ds(base, rem)],
                                dstbuf.at[pl.ds(0, rem)])
                w = scan_groups(w, rem // LANES, lo)

            @pl.when(w > 0)
            def _():
                fire(w)

            plsc.subcore_barrier()
            for off, sz in _splits(RS, min(ZR, B)):
                pltpu.sync_copy(acc_sh.at[pl.ds(s * RS + off, sz)],
                                rows_v.at[pl.ds(0, sz)])
                pltpu.sync_copy(rows_v.at[pl.ds(0, sz)],
                                out_hbm.at[pl.ds(lo + s * RS + off, sz)])

    return spmm_k


def _tc_phase1(x, W1, deg_parts3, NP, BR=1024):
    N, INC = x.shape
    HID = W1.shape[1]
    grid = (NP // BR,)

    def body(x_ref, w_ref, dp_ref, h_ref, dinv_ref):
        dsum = dp_ref[0] + dp_ref[1] + 1.0
        dinv = lax.rsqrt(dsum)
        dinv_ref[...] = dinv
        h = jnp.dot(x_ref[...], w_ref[...],
                    preferred_element_type=jnp.float32)
        h_ref[...] = h * dinv

    return pl.pallas_call(
        body,
        grid=grid,
        in_specs=[
            pl.BlockSpec((BR, INC), lambda i: (i, 0)),
            pl.BlockSpec((INC, HID), lambda i: (0, 0)),
            pl.BlockSpec((NC, BR, 1), lambda i: (0, i, 0)),
        ],
        out_specs=[
            pl.BlockSpec((BR, HID), lambda i: (i, 0)),
            pl.BlockSpec((BR, 1), lambda i: (i, 0)),
        ],
        out_shape=[
            jax.ShapeDtypeStruct((N, HID), jnp.float32),
            jax.ShapeDtypeStruct((NP, 1), jnp.float32),
        ],
    )(x, W1, deg_parts3)


def _tc_phase2(acc1, h1p, dinv, W2, b1, BR=1024):
    N, HID = h1p.shape
    OUT = W2.shape[1]
    NP = dinv.shape[0]
    grid = (NP // BR,)

    def body(a_ref, h_ref, d_ref, w_ref, b_ref, o_ref):
        t = (a_ref[...] + h_ref[...]) * d_ref[...] + b_ref[...]
        h2 = jnp.dot(t, w_ref[...], preferred_element_type=jnp.float32)
        o_ref[...] = h2 * d_ref[...]

    return pl.pallas_call(
        body,
        grid=grid,
        in_specs=[
            pl.BlockSpec((BR, HID), lambda i: (i, 0)),
            pl.BlockSpec((BR, HID), lambda i: (i, 0)),
            pl.BlockSpec((BR, 1), lambda i: (i, 0)),
            pl.BlockSpec((HID, OUT), lambda i: (0, 0)),
            pl.BlockSpec((1, HID), lambda i: (0, 0)),
        ],
        out_specs=pl.BlockSpec((BR, OUT), lambda i: (i, 0)),
        out_shape=jax.ShapeDtypeStruct((N, OUT), jnp.float32),
    )(acc1, h1p, dinv, W2, b1)


def _tc_phase3(acc2, h2p, dinv, b2, BR=1024):
    N, OUT = h2p.shape
    NP = dinv.shape[0]
    grid = (NP // BR,)

    def body(a_ref, h_ref, d_ref, b_ref, o_ref):
        o_ref[...] = (a_ref[...] + h_ref[...]) * d_ref[...] + b_ref[...]

    return pl.pallas_call(
        body,
        grid=grid,
        in_specs=[
            pl.BlockSpec((BR, OUT), lambda i: (i, 0)),
            pl.BlockSpec((BR, OUT), lambda i: (i, 0)),
            pl.BlockSpec((BR, 1), lambda i: (i, 0)),
            pl.BlockSpec((1, OUT), lambda i: (0, 0)),
        ],
        out_specs=pl.BlockSpec((BR, OUT), lambda i: (i, 0)),
        out_shape=jax.ShapeDtypeStruct((N, OUT), jnp.float32),
    )(acc2, h2p, dinv, b2)


def kernel(x, edge_index, W1, b1, W2, b2):
    N, INC = x.shape
    E = edge_index.shape[1]
    HID = W1.shape[1]
    OUT = W2.shape[1]
    NP = ((N + 1023) // 1024) * 1024
    CR = 12800
    OUTROWS = -(-N // CR) * CR

    ei = edge_index.astype(jnp.int32)
    src = ei[0]
    dst = ei[1]

    deg_parts = _make_deg_kernel(E, NP)(dst)
    deg_parts3 = deg_parts.reshape(NC, NP, 1)

    h1p, dinv = _tc_phase1(x, W1, deg_parts3, NP)

    spmm = _make_spmm_kernel(HID, E, CR, OUTROWS // CR, OUTROWS)
    acc1 = spmm(h1p, src, dst)

    W2p = jnp.pad(W2, ((0, 0), (0, HID - OUT)))
    h2p = _tc_phase2(acc1[:N], h1p, dinv, W2p, b1.reshape(1, HID))

    acc2 = spmm(h2p, src, dst)
    b2p = jnp.pad(b2.reshape(1, OUT), ((0, 0), (0, HID - OUT)))
    z = _tc_phase3(acc2[:N], h2p, dinv, b2p)
    return z[:, :OUT]
